# Initial kernel scaffold; baseline (speedup 1.0000x reference)
import jax
import jax.numpy as jnp
from jax.experimental import pallas as pl


def kernel(data, channels):
    raise NotImplementedError("write your pallas kernel here")


# speedup vs baseline: 1.6702x; 1.6702x over previous
import jax
import jax.numpy as jnp
from jax.experimental import pallas as pl
from jax.experimental.pallas import tpu as pltpu

_BR = 512


def _body(ch_ref, x_ref, o_ref):
    x = x_ref[...]
    cols = jax.lax.broadcasted_iota(jnp.int32, (1, x.shape[1]), 1)
    ch = ch_ref[...]
    mask = jnp.any(ch == cols, axis=0, keepdims=True)
    o_ref[...] = jnp.where(mask, jnp.tanh(x), x)


def kernel(data, channels):
    n, d = data.shape
    ch2 = channels.reshape(-1, 1)
    grid = (n // _BR,)
    return pl.pallas_call(
        _body,
        grid=grid,
        in_specs=[
            pl.BlockSpec((ch2.shape[0], 1), lambda i: (0, 0)),
            pl.BlockSpec((_BR, d), lambda i: (i, 0)),
        ],
        out_specs=pl.BlockSpec((_BR, d), lambda i: (i, 0)),
        out_shape=jax.ShapeDtypeStruct((n, d), data.dtype),
        compiler_params=pltpu.CompilerParams(
            dimension_semantics=("arbitrary",),
        ),
    )(ch2, data)


# speedup vs baseline: 3.3822x; 2.0250x over previous
import jax
import jax.numpy as jnp
from jax.experimental import pallas as pl
from jax.experimental.pallas import tpu as pltpu

_BR = 512


def _body(ch_ref, x_ref, o_ref, mask_ref):
    @pl.when(pl.program_id(0) == 0)
    def _build_mask():
        cols = jax.lax.broadcasted_iota(jnp.int32, (1, mask_ref.shape[1]), 1)
        ch = ch_ref[...]
        m = jnp.any(ch == cols, axis=0, keepdims=True).astype(jnp.float32)
        mask_ref[...] = jnp.broadcast_to(m, mask_ref.shape)

    x = x_ref[...]
    m = mask_ref[...]
    o_ref[...] = x + m * (jnp.tanh(x) - x)


def kernel(data, channels):
    n, d = data.shape
    ch2 = channels.reshape(-1, 1)
    grid = (n // _BR,)
    return pl.pallas_call(
        _body,
        grid=grid,
        in_specs=[
            pl.BlockSpec((ch2.shape[0], 1), lambda i: (0, 0)),
            pl.BlockSpec((_BR, d), lambda i: (i, 0)),
        ],
        out_specs=pl.BlockSpec((_BR, d), lambda i: (i, 0)),
        out_shape=jax.ShapeDtypeStruct((n, d), data.dtype),
        scratch_shapes=[pltpu.VMEM((_BR, d), jnp.float32)],
        compiler_params=pltpu.CompilerParams(
            dimension_semantics=("arbitrary",),
        ),
    )(ch2, data)
